# Initial kernel scaffold; baseline (speedup 1.0000x reference)
import jax
import jax.numpy as jnp
from jax.experimental import pallas as pl


def kernel(C, K, tie_kb_keys, tie_kb_values, Q_w, top_k, temperature):
    raise NotImplementedError("write your pallas kernel here")


# speedup vs baseline: 3.4328x; 3.4328x over previous
import functools

import jax
import jax.numpy as jnp
from jax import lax
from jax.experimental import pallas as pl
from jax.experimental.pallas import tpu as pltpu
from jax.experimental.pallas import tpu_sc as plsc

N = 1024
D_C = 512
D_K = 256
D_T = 512
N_KB = 100000
TOPK = 32

R = 256
CK = 2048
NCH = 49
NKB_PAD = NCH * CK
NEG = -3.0e38


def _topk_kernel(cs_ref, qw_ref, keys_ref, idx_out_ref, w_out_ref,
                 q_ref, s_ref, top_ref, tidx_ref, done_ref):
    j = pl.program_id(1)

    @pl.when(j == 0)
    def _init():
        q_ref[...] = lax.dot_general(
            cs_ref[...], qw_ref[...], (((1,), (1,)), ((), ())),
            preferred_element_type=jnp.float32,
            precision=lax.Precision.HIGHEST)
        top_ref[...] = jnp.full((R, TOPK), NEG, jnp.float32)
        tidx_ref[...] = jnp.zeros((R, TOPK), jnp.int32)

    s = lax.dot_general(
        q_ref[...], keys_ref[...], (((1,), (1,)), ((), ())),
        preferred_element_type=jnp.float32,
        precision=lax.Precision.HIGHEST)
    gcol = j * CK + lax.broadcasted_iota(jnp.int32, (R, CK), 1)
    s_ref[...] = jnp.where(gcol < N_KB, s, NEG)
    done_ref[0] = 0

    def _extract(_, carry):
        @pl.when(done_ref[0] == 0)
        def _():
            sc = s_ref[...]
            m = jnp.max(sc, axis=1, keepdims=True)
            t = top_ref[...]
            min32 = jnp.min(t, axis=1, keepdims=True)
            need = m > min32
            any_need = jnp.max(m - min32) > 0.0

            @pl.when(jnp.logical_not(any_need))
            def _():
                done_ref[0] = 1

            @pl.when(any_need)
            def _():
                col = lax.broadcasted_iota(jnp.int32, (R, CK), 1)
                ismax = sc == m
                am = jnp.min(jnp.where(ismax, col, jnp.int32(2**30)),
                             axis=1, keepdims=True)
                s_ref[...] = jnp.where(col == am, NEG, sc)
                lane = lax.broadcasted_iota(jnp.int32, (R, TOPK), 1)
                ismin = t == min32
                pos = jnp.min(jnp.where(ismin, lane, jnp.int32(2**30)),
                              axis=1, keepdims=True)
                sel = (lane == pos) & need
                top_ref[...] = jnp.where(sel, m, t)
                tidx_ref[...] = jnp.where(sel, j * CK + am, tidx_ref[...])
        return carry

    lax.fori_loop(0, TOPK, _extract, 0)

    @pl.when(j == NCH - 1)
    def _finish():
        t = top_ref[...]
        mx = jnp.max(t, axis=1, keepdims=True)
        e = jnp.exp(t - mx)
        w_out_ref[...] = e / jnp.sum(e, axis=1, keepdims=True)
        idx_out_ref[...] = tidx_ref[...]


@jax.jit
def _topk_call(cs, qw, keys_pad):
    return pl.pallas_call(
        _topk_kernel,
        grid=(N // R, NCH),
        in_specs=[
            pl.BlockSpec((R, D_C), lambda i, j: (i, 0)),
            pl.BlockSpec((D_T, D_C), lambda i, j: (0, 0)),
            pl.BlockSpec((CK, D_T), lambda i, j: (j, 0)),
        ],
        out_specs=[
            pl.BlockSpec((R, TOPK), lambda i, j: (i, 0)),
            pl.BlockSpec((R, TOPK), lambda i, j: (i, 0)),
        ],
        out_shape=[
            jax.ShapeDtypeStruct((N, TOPK), jnp.int32),
            jax.ShapeDtypeStruct((N, TOPK), jnp.float32),
        ],
        scratch_shapes=[
            pltpu.VMEM((R, D_T), jnp.float32),
            pltpu.VMEM((R, CK), jnp.float32),
            pltpu.VMEM((R, TOPK), jnp.float32),
            pltpu.VMEM((R, TOPK), jnp.int32),
            pltpu.SMEM((1,), jnp.int32),
        ],
        compiler_params=pltpu.CompilerParams(
            dimension_semantics=("arbitrary", "arbitrary")),
    )(cs, qw, keys_pad)


_NW = 32
_B = N * TOPK
_BPW = _B // _NW
_GCH = 128
_NGC = _BPW // _GCH


def _gather_body(idx_hbm, table_hbm, out_hbm, idx_v, rows_v, sem):
    wid = lax.axis_index("s") * 2 + lax.axis_index("c")
    base = wid * _BPW

    def body(c, carry):
        off = base + c * _GCH
        pltpu.sync_copy(idx_hbm.at[pl.ds(off, _GCH)], idx_v)
        pltpu.async_copy(table_hbm.at[idx_v], rows_v, sem).wait()
        pltpu.sync_copy(rows_v, out_hbm.at[pl.ds(off, _GCH)])
        return carry

    lax.fori_loop(0, _NGC, body, 0)


@jax.jit
def _gather_call(idx_flat, table):
    f = functools.partial(
        pl.kernel,
        mesh=plsc.VectorSubcoreMesh(core_axis_name="c", subcore_axis_name="s"),
        out_type=jax.ShapeDtypeStruct((_B, D_T), jnp.float32),
        scratch_types=[
            pltpu.VMEM((_GCH,), jnp.int32),
            pltpu.VMEM((_GCH, D_T), jnp.float32),
            pltpu.SemaphoreType.DMA,
        ],
    )(_gather_body)
    return f(idx_flat, table)


RB = 64


def _fuse_kernel(c_ref, k_ref, w_ref, g_ref, o_ref):
    w = w_ref[...]
    acc = jnp.zeros((RB, D_T), jnp.float32)
    for kk in range(TOPK):
        acc = acc + w[:, kk:kk + 1] * g_ref[:, kk, :]
    o_ref[...] = jnp.concatenate([
        jnp.maximum(c_ref[...], 0.0),
        jnp.maximum(k_ref[...], 0.0),
        jnp.maximum(0.5 * acc, 0.0),
    ], axis=1)


@jax.jit
def _fuse_call(C, K, w, g):
    return pl.pallas_call(
        _fuse_kernel,
        grid=(N // RB,),
        in_specs=[
            pl.BlockSpec((RB, D_C), lambda i: (i, 0)),
            pl.BlockSpec((RB, D_K), lambda i: (i, 0)),
            pl.BlockSpec((RB, TOPK), lambda i: (i, 0)),
            pl.BlockSpec((RB, TOPK, D_T), lambda i: (i, 0, 0)),
        ],
        out_specs=pl.BlockSpec((RB, D_C + D_K + D_T), lambda i: (i, 0)),
        out_shape=jax.ShapeDtypeStruct((N, D_C + D_K + D_T), jnp.float32),
    )(C, K, w, g)


def kernel(C, K, tie_kb_keys, tie_kb_values, Q_w, top_k, temperature):
    del top_k
    cs = C / temperature
    keys_pad = jnp.pad(tie_kb_keys, ((0, NKB_PAD - N_KB), (0, 0)))
    idx, w = _topk_call(cs, Q_w, keys_pad)
    g = _gather_call(idx.reshape(-1), tie_kb_values)
    g = g.reshape(N, TOPK, D_T)
    return _fuse_call(C, K, w, g)


# speedup vs baseline: 4.4578x; 1.2986x over previous
import functools

import jax
import jax.numpy as jnp
from jax import lax
from jax.experimental import pallas as pl
from jax.experimental.pallas import tpu as pltpu
from jax.experimental.pallas import tpu_sc as plsc

N = 1024
D_C = 512
D_K = 256
D_T = 512
N_KB = 100000
TOPK = 32

R = 256
CK = 2048
NCH = 49
NKB_PAD = NCH * CK
NEG = -3.0e38


def _topk_kernel(cs_ref, qw_ref, keys_ref, idx_out_ref, w_out_ref,
                 q_ref, s_ref, top_ref, tidx_ref, done_ref):
    j = pl.program_id(1)

    @pl.when(j == 0)
    def _init():
        q_ref[...] = lax.dot_general(
            cs_ref[...], qw_ref[...], (((1,), (1,)), ((), ())),
            preferred_element_type=jnp.float32)
        top_ref[...] = jnp.full((R, TOPK), NEG, jnp.float32)
        tidx_ref[...] = jnp.zeros((R, TOPK), jnp.int32)

    s = lax.dot_general(
        q_ref[...], keys_ref[...], (((1,), (1,)), ((), ())),
        preferred_element_type=jnp.float32)
    gcol = j * CK + lax.broadcasted_iota(jnp.int32, (R, CK), 1)
    s_ref[...] = jnp.where(gcol < N_KB, s, NEG)
    done_ref[0] = 0

    def _extract(_, carry):
        @pl.when(done_ref[0] == 0)
        def _():
            sc = s_ref[...]
            m = jnp.max(sc, axis=1, keepdims=True)
            t = top_ref[...]
            min32 = jnp.min(t, axis=1, keepdims=True)
            need = m > min32
            any_need = jnp.max(m - min32) > 0.0

            @pl.when(jnp.logical_not(any_need))
            def _():
                done_ref[0] = 1

            @pl.when(any_need)
            def _():
                col = lax.broadcasted_iota(jnp.int32, (R, CK), 1)
                ismax = sc == m
                am = jnp.min(jnp.where(ismax, col, jnp.int32(2**30)),
                             axis=1, keepdims=True)
                s_ref[...] = jnp.where(col == am, NEG, sc)
                lane = lax.broadcasted_iota(jnp.int32, (R, TOPK), 1)
                ismin = t == min32
                pos = jnp.min(jnp.where(ismin, lane, jnp.int32(2**30)),
                              axis=1, keepdims=True)
                sel = (lane == pos) & need
                top_ref[...] = jnp.where(sel, m, t)
                tidx_ref[...] = jnp.where(sel, j * CK + am, tidx_ref[...])
        return carry

    lax.fori_loop(0, TOPK, _extract, 0)

    @pl.when(j == NCH - 1)
    def _finish():
        t = top_ref[...]
        mx = jnp.max(t, axis=1, keepdims=True)
        e = jnp.exp(t - mx)
        w_out_ref[...] = e / jnp.sum(e, axis=1, keepdims=True)
        idx_out_ref[...] = tidx_ref[...]


@jax.jit
def _topk_call(cs, qw, keys_pad):
    return pl.pallas_call(
        _topk_kernel,
        grid=(N // R, NCH),
        in_specs=[
            pl.BlockSpec((R, D_C), lambda i, j: (i, 0)),
            pl.BlockSpec((D_T, D_C), lambda i, j: (0, 0)),
            pl.BlockSpec((CK, D_T), lambda i, j: (j, 0)),
        ],
        out_specs=[
            pl.BlockSpec((R, TOPK), lambda i, j: (i, 0)),
            pl.BlockSpec((R, TOPK), lambda i, j: (i, 0)),
        ],
        out_shape=[
            jax.ShapeDtypeStruct((N, TOPK), jnp.int32),
            jax.ShapeDtypeStruct((N, TOPK), jnp.float32),
        ],
        scratch_shapes=[
            pltpu.VMEM((R, D_T), jnp.float32),
            pltpu.VMEM((R, CK), jnp.float32),
            pltpu.VMEM((R, TOPK), jnp.float32),
            pltpu.VMEM((R, TOPK), jnp.int32),
            pltpu.SMEM((1,), jnp.int32),
        ],
        compiler_params=pltpu.CompilerParams(
            dimension_semantics=("arbitrary", "arbitrary")),
    )(cs, qw, keys_pad)


_NW = 32
_B = N * TOPK
_BPW = _B // _NW
_GCH = 128
_NGC = _BPW // _GCH


def _gather_body(idx_hbm, table_hbm, out_hbm, idx_v, rows_v, sem):
    wid = lax.axis_index("s") * 2 + lax.axis_index("c")
    base = wid * _BPW

    def body(c, carry):
        off = base + c * _GCH
        pltpu.sync_copy(idx_hbm.at[pl.ds(off, _GCH)], idx_v)
        pltpu.async_copy(table_hbm.at[idx_v], rows_v, sem).wait()
        pltpu.sync_copy(rows_v, out_hbm.at[pl.ds(off, _GCH)])
        return carry

    lax.fori_loop(0, _NGC, body, 0)


@jax.jit
def _gather_call(idx_flat, table):
    f = functools.partial(
        pl.kernel,
        mesh=plsc.VectorSubcoreMesh(core_axis_name="c", subcore_axis_name="s"),
        out_type=jax.ShapeDtypeStruct((_B, D_T), jnp.float32),
        scratch_types=[
            pltpu.VMEM((_GCH,), jnp.int32),
            pltpu.VMEM((_GCH, D_T), jnp.float32),
            pltpu.SemaphoreType.DMA,
        ],
    )(_gather_body)
    return f(idx_flat, table)


RB = 64


def _fuse_kernel(c_ref, k_ref, w_ref, g_ref, o_ref):
    w = w_ref[...]
    acc = jnp.zeros((RB, D_T), jnp.float32)
    for kk in range(TOPK):
        acc = acc + w[:, kk:kk + 1] * g_ref[:, kk, :]
    o_ref[...] = jnp.concatenate([
        jnp.maximum(c_ref[...], 0.0),
        jnp.maximum(k_ref[...], 0.0),
        jnp.maximum(0.5 * acc, 0.0),
    ], axis=1)


@jax.jit
def _fuse_call(C, K, w, g):
    return pl.pallas_call(
        _fuse_kernel,
        grid=(N // RB,),
        in_specs=[
            pl.BlockSpec((RB, D_C), lambda i: (i, 0)),
            pl.BlockSpec((RB, D_K), lambda i: (i, 0)),
            pl.BlockSpec((RB, TOPK), lambda i: (i, 0)),
            pl.BlockSpec((RB, TOPK, D_T), lambda i: (i, 0, 0)),
        ],
        out_specs=pl.BlockSpec((RB, D_C + D_K + D_T), lambda i: (i, 0)),
        out_shape=jax.ShapeDtypeStruct((N, D_C + D_K + D_T), jnp.float32),
    )(C, K, w, g)


def kernel(C, K, tie_kb_keys, tie_kb_values, Q_w, top_k, temperature):
    del top_k
    cs = C / temperature
    keys_pad = jnp.pad(tie_kb_keys, ((0, NKB_PAD - N_KB), (0, 0)))
    idx, w = _topk_call(cs, Q_w, keys_pad)
    g = _gather_call(idx.reshape(-1), tie_kb_values)
    g = g.reshape(N, TOPK, D_T)
    return _fuse_call(C, K, w, g)
